# baseline (speedup 1.0000x reference)
import functools

import jax
import jax.numpy as jnp
from jax.experimental import pallas as pl
from jax.experimental.pallas import tpu as pltpu

_ANCHORS = ((116.0, 90.0), (156.0, 198.0), (373.0, 326.0))
_IMG = 512.0
_IGNORE_THR = 0.5
_LXY = 2.5
_LCONF = 5.0
_EPS = 1e-12
_TOP = 1.0 - 1e-12


def _body(t_ref, tv_ref, hbm_ref, out_ref, bufs, rows, dsem, rsem,
          *, in_h, in_w, n_total, bs, bb):
    nchunk = 3 * bs // bb

    def row_copy(b):
        gj = jnp.floor(t_ref[b, 0, 2] * in_h).astype(jnp.int32)
        return pltpu.make_async_copy(
            hbm_ref.at[b, :, pl.ds(gj, 1), :], rows.at[b], rsem)

    for b in range(bs):
        row_copy(b).start()

    ng = bs // bb

    def chunk_copy(g, a):
        return pltpu.make_async_copy(
            hbm_ref.at[pl.ds(g * bb, bb), 3 * a + 2, :, :],
            bufs.at[pl.ds(g * bb, bb), a], dsem.at[3 * g + a])

    for g in range(ng):
        for a in range(3):
            chunk_copy(g, a).start()

    for b in range(bs):
        row_copy(b).wait()

    tv = tv_ref[:, 0, :]
    gx = tv[:, 1:2] * in_w
    gy = tv[:, 2:3] * in_h
    gw = tv[:, 3:4] * in_w
    gh = tv[:, 4:5] * in_h
    fx = jnp.floor(gx)
    gi = fx.astype(jnp.int32)
    tx = gx - fx
    ty = gy - jnp.floor(gy)

    stride_w = _IMG / in_w
    stride_h = _IMG / in_h
    ious = []
    for aw, ah in _ANCHORS:
        aw = aw / stride_w
        ah = ah / stride_h
        inter = (jnp.maximum(jnp.minimum(gw, aw), 0.0)
                 * jnp.maximum(jnp.minimum(gh, ah), 0.0))
        union = gw * gh + aw * ah - inter + 1e-16
        ious.append(inter / union)
    best = jnp.zeros_like(gi)
    bv = ious[0]
    best = jnp.where(ious[1] > bv, jnp.int32(1), best)
    bv = jnp.maximum(bv, ious[1])
    best = jnp.where(ious[2] > bv, jnp.int32(2), best)

    v = rows[:, :, 0, :]
    lane = jax.lax.broadcasted_iota(jnp.int32, v.shape, 2)
    vals = jnp.sum(jnp.where(lane == gi[:, :, None], v, 0.0), axis=2)

    sparse = jnp.zeros_like(gx)
    zx = jnp.zeros_like(gx)
    zy = jnp.zeros_like(gx)
    for a in range(3):
        zc = vals[:, 3 * a + 2:3 * a + 3]
        sparse -= jnp.where(ious[a] > _IGNORE_THR,
                            0.5 * _LCONF * jnp.log1p(jnp.exp(zc)), 0.0)
        p_t = jnp.clip(jax.nn.sigmoid(zc), _EPS, _TOP)
        sparse += jnp.where(a == best, -_LCONF * jnp.log(p_t), 0.0)
        zx += jnp.where(a == best, vals[:, 3 * a:3 * a + 1], 0.0)
        zy += jnp.where(a == best, vals[:, 3 * a + 1:3 * a + 2], 0.0)
    for z_v, t_v in ((zx, tx), (zy, ty)):
        p_v = jnp.clip(jax.nn.sigmoid(z_v), _EPS, _TOP)
        sparse += -_LXY * (t_v * jnp.log(p_v)
                           + (1.0 - t_v) * jnp.log(1.0 - p_v))

    acc = jnp.zeros((8, in_w), jnp.float32)
    for g in range(ng):
        for a in range(3):
            chunk_copy(g, a).wait()
        z = bufs[g * bb:(g + 1) * bb]
        l = jnp.log1p(jnp.exp(z))
        acc = acc + jnp.sum(
            l.reshape(bb * 3 * in_h // 8, 8, in_w), axis=0)

    total = 0.5 * _LCONF * jnp.sum(acc) + jnp.sum(sparse)
    out_ref[0, 0] = total / n_total


def kernel(input, targets):
    bs, ch, in_h, in_w = input.shape
    bb = 4
    body = functools.partial(_body, in_h=in_h, in_w=in_w,
                             n_total=float(bs * 3 * in_h * in_w),
                             bs=bs, bb=bb)
    out = pl.pallas_call(
        body,
        grid=(1,),
        in_specs=[
            pl.BlockSpec(targets.shape, lambda i: (0, 0, 0),
                         memory_space=pltpu.SMEM),
            pl.BlockSpec(targets.shape, lambda i: (0, 0, 0)),
            pl.BlockSpec(memory_space=pl.ANY),
        ],
        out_specs=pl.BlockSpec((1, 1), lambda i: (0, 0),
                               memory_space=pltpu.SMEM),
        out_shape=jax.ShapeDtypeStruct((1, 1), jnp.float32),
        scratch_shapes=[
            pltpu.VMEM((bs, 3, in_h, in_w), jnp.float32),
            pltpu.VMEM((bs, ch, 1, in_w), jnp.float32),
            pltpu.SemaphoreType.DMA((3 * bs // bb,)),
            pltpu.SemaphoreType.DMA,
        ],
    )(targets, targets, input)
    return out[0, 0]


# speedup vs baseline: 1.0012x; 1.0012x over previous
import functools

import jax
import jax.numpy as jnp
from jax.experimental import pallas as pl
from jax.experimental.pallas import tpu as pltpu

_ANCHORS = ((116.0, 90.0), (156.0, 198.0), (373.0, 326.0))
_IMG = 512.0
_IGNORE_THR = 0.5
_LXY = 2.5
_LCONF = 5.0
_EPS = 1e-12
_TOP = 1.0 - 1e-12


def _body(t_ref, tv_ref, hbm_ref, out_ref, bufs, rows, dsem, rsem,
          *, in_h, in_w, n_total, bs, bb):
    nchunk = 3 * bs // bb

    def row_copy(b):
        gj = jnp.floor(t_ref[b, 0, 2] * in_h).astype(jnp.int32)
        return pltpu.make_async_copy(
            hbm_ref.at[b, :, pl.ds(gj, 1), :], rows.at[b], rsem)

    for b in range(bs):
        row_copy(b).start()

    ng = bs // bb

    def chunk_copy(g, a):
        return pltpu.make_async_copy(
            hbm_ref.at[pl.ds(g * bb, bb), 3 * a + 2, :, :],
            bufs.at[pl.ds(g * bb, bb), a], dsem.at[3 * g + a])

    for g in range(ng):
        for a in range(3):
            chunk_copy(g, a).start()

    acc = jnp.zeros((8, in_w), jnp.float32)
    for g in range(ng):
        for a in range(3):
            chunk_copy(g, a).wait()
        z = bufs[g * bb:(g + 1) * bb]
        l = jnp.log1p(jnp.exp(z))
        acc = acc + jnp.sum(
            l.reshape(bb * 3 * in_h // 8, 8, in_w), axis=0)

    for b in range(bs):
        row_copy(b).wait()

    tv = tv_ref[:, 0, :]
    gx = tv[:, 1:2] * in_w
    gy = tv[:, 2:3] * in_h
    gw = tv[:, 3:4] * in_w
    gh = tv[:, 4:5] * in_h
    fx = jnp.floor(gx)
    gi = fx.astype(jnp.int32)
    tx = gx - fx
    ty = gy - jnp.floor(gy)

    stride_w = _IMG / in_w
    stride_h = _IMG / in_h
    ious = []
    for aw, ah in _ANCHORS:
        aw = aw / stride_w
        ah = ah / stride_h
        inter = (jnp.maximum(jnp.minimum(gw, aw), 0.0)
                 * jnp.maximum(jnp.minimum(gh, ah), 0.0))
        union = gw * gh + aw * ah - inter + 1e-16
        ious.append(inter / union)
    best = jnp.zeros_like(gi)
    bv = ious[0]
    best = jnp.where(ious[1] > bv, jnp.int32(1), best)
    bv = jnp.maximum(bv, ious[1])
    best = jnp.where(ious[2] > bv, jnp.int32(2), best)

    v = rows[:, :, 0, :]
    lane = jax.lax.broadcasted_iota(jnp.int32, v.shape, 2)
    vals = jnp.sum(jnp.where(lane == gi[:, :, None], v, 0.0), axis=2)

    sparse = jnp.zeros_like(gx)
    zx = jnp.zeros_like(gx)
    zy = jnp.zeros_like(gx)
    for a in range(3):
        zc = vals[:, 3 * a + 2:3 * a + 3]
        sparse -= jnp.where(ious[a] > _IGNORE_THR,
                            0.5 * _LCONF * jnp.log1p(jnp.exp(zc)), 0.0)
        p_t = jnp.clip(jax.nn.sigmoid(zc), _EPS, _TOP)
        sparse += jnp.where(a == best, -_LCONF * jnp.log(p_t), 0.0)
        zx += jnp.where(a == best, vals[:, 3 * a:3 * a + 1], 0.0)
        zy += jnp.where(a == best, vals[:, 3 * a + 1:3 * a + 2], 0.0)
    for z_v, t_v in ((zx, tx), (zy, ty)):
        p_v = jnp.clip(jax.nn.sigmoid(z_v), _EPS, _TOP)
        sparse += -_LXY * (t_v * jnp.log(p_v)
                           + (1.0 - t_v) * jnp.log(1.0 - p_v))

    total = 0.5 * _LCONF * jnp.sum(acc) + jnp.sum(sparse)
    out_ref[0, 0] = total / n_total


def kernel(input, targets):
    bs, ch, in_h, in_w = input.shape
    bb = 8
    body = functools.partial(_body, in_h=in_h, in_w=in_w,
                             n_total=float(bs * 3 * in_h * in_w),
                             bs=bs, bb=bb)
    out = pl.pallas_call(
        body,
        grid=(1,),
        in_specs=[
            pl.BlockSpec(targets.shape, lambda i: (0, 0, 0),
                         memory_space=pltpu.SMEM),
            pl.BlockSpec(targets.shape, lambda i: (0, 0, 0)),
            pl.BlockSpec(memory_space=pl.ANY),
        ],
        out_specs=pl.BlockSpec((1, 1), lambda i: (0, 0),
                               memory_space=pltpu.SMEM),
        out_shape=jax.ShapeDtypeStruct((1, 1), jnp.float32),
        scratch_shapes=[
            pltpu.VMEM((bs, 3, in_h, in_w), jnp.float32),
            pltpu.VMEM((bs, ch, 1, in_w), jnp.float32),
            pltpu.SemaphoreType.DMA((3 * bs // bb,)),
            pltpu.SemaphoreType.DMA,
        ],
    )(targets, targets, input)
    return out[0, 0]


# speedup vs baseline: 1.0853x; 1.0841x over previous
import functools

import jax
import jax.numpy as jnp
from jax.experimental import pallas as pl
from jax.experimental.pallas import tpu as pltpu

_ANCHORS = ((116.0, 90.0), (156.0, 198.0), (373.0, 326.0))
_IMG = 512.0
_IGNORE_THR = 0.5
_LXY = 2.5
_LCONF = 5.0
_EPS = 1e-12
_TOP = 1.0 - 1e-12


def _body(t_ref, tv_ref, hbm_ref, out_ref, bufs, rows, dsem, rsem,
          *, in_h, in_w, n_total, bs, bb):
    nchunk = 3 * bs // bb

    def row_copy(b):
        gj = jnp.floor(t_ref[b, 0, 2] * in_h).astype(jnp.int32)
        return pltpu.make_async_copy(
            hbm_ref.at[b, :, pl.ds(gj, 1), :], rows.at[b], rsem)

    for b in range(bs):
        row_copy(b).start()

    ng = bs // bb

    def chunk_copy(g, a):
        return pltpu.make_async_copy(
            hbm_ref.at[pl.ds(g * bb, bb), 3 * a + 2, :, :],
            bufs.at[pl.ds(g * bb, bb), a], dsem.at[3 * g + a])

    for g in range(ng):
        for a in range(3):
            chunk_copy(g, a).start()

    acc = jnp.zeros((8, in_w), jnp.float32)
    for g in range(ng):
        for a in range(3):
            chunk_copy(g, a).wait()
        z = bufs[g * bb:(g + 1) * bb]
        l = jnp.log1p(jnp.exp(z))
        acc = acc + jnp.sum(
            l.reshape(bb * 3 * in_h // 8, 8, in_w), axis=0)

    for b in range(bs):
        row_copy(b).wait()

    tv = tv_ref[:, 0, :]
    gx = tv[:, 1:2] * in_w
    gy = tv[:, 2:3] * in_h
    gw = tv[:, 3:4] * in_w
    gh = tv[:, 4:5] * in_h
    fx = jnp.floor(gx)
    gi = fx.astype(jnp.int32)
    tx = gx - fx
    ty = gy - jnp.floor(gy)

    stride_w = _IMG / in_w
    stride_h = _IMG / in_h
    ious = []
    for aw, ah in _ANCHORS:
        aw = aw / stride_w
        ah = ah / stride_h
        inter = (jnp.maximum(jnp.minimum(gw, aw), 0.0)
                 * jnp.maximum(jnp.minimum(gh, ah), 0.0))
        union = gw * gh + aw * ah - inter + 1e-16
        ious.append(inter / union)
    best = jnp.zeros_like(gi)
    bv = ious[0]
    best = jnp.where(ious[1] > bv, jnp.int32(1), best)
    bv = jnp.maximum(bv, ious[1])
    best = jnp.where(ious[2] > bv, jnp.int32(2), best)

    v = rows[:, :, 0, :]
    lane = jax.lax.broadcasted_iota(jnp.int32, v.shape, 2)
    vals = jnp.sum(jnp.where(lane == gi[:, :, None], v, 0.0), axis=2)

    sparse = jnp.zeros_like(gx)
    zx = jnp.zeros_like(gx)
    zy = jnp.zeros_like(gx)
    for a in range(3):
        zc = vals[:, 3 * a + 2:3 * a + 3]
        sparse -= jnp.where(ious[a] > _IGNORE_THR,
                            0.5 * _LCONF * jnp.log1p(jnp.exp(zc)), 0.0)
        p_t = jnp.clip(jax.nn.sigmoid(zc), _EPS, _TOP)
        sparse += jnp.where(a == best, -_LCONF * jnp.log(p_t), 0.0)
        zx += jnp.where(a == best, vals[:, 3 * a:3 * a + 1], 0.0)
        zy += jnp.where(a == best, vals[:, 3 * a + 1:3 * a + 2], 0.0)
    for z_v, t_v in ((zx, tx), (zy, ty)):
        p_v = jnp.clip(jax.nn.sigmoid(z_v), _EPS, _TOP)
        sparse += -_LXY * (t_v * jnp.log(p_v)
                           + (1.0 - t_v) * jnp.log(1.0 - p_v))

    total = 0.5 * _LCONF * jnp.sum(acc) + jnp.sum(sparse)
    out_ref[0, 0] = total / n_total


def kernel(input, targets):
    bs, ch, in_h, in_w = input.shape
    bb = 2
    body = functools.partial(_body, in_h=in_h, in_w=in_w,
                             n_total=float(bs * 3 * in_h * in_w),
                             bs=bs, bb=bb)
    out = pl.pallas_call(
        body,
        grid=(1,),
        in_specs=[
            pl.BlockSpec(targets.shape, lambda i: (0, 0, 0),
                         memory_space=pltpu.SMEM),
            pl.BlockSpec(targets.shape, lambda i: (0, 0, 0)),
            pl.BlockSpec(memory_space=pl.ANY),
        ],
        out_specs=pl.BlockSpec((1, 1), lambda i: (0, 0),
                               memory_space=pltpu.SMEM),
        out_shape=jax.ShapeDtypeStruct((1, 1), jnp.float32),
        scratch_shapes=[
            pltpu.VMEM((bs, 3, in_h, in_w), jnp.float32),
            pltpu.VMEM((bs, ch, 1, in_w), jnp.float32),
            pltpu.SemaphoreType.DMA((3 * bs // bb,)),
            pltpu.SemaphoreType.DMA,
        ],
    )(targets, targets, input)
    return out[0, 0]
